# baseline (speedup 1.0000x reference)
import dataclasses
import functools
import math

import jax
import jax.numpy as jnp
from jax import lax
from jax.experimental import pallas as pl
from jax.experimental.pallas import tpu as pltpu
from jax.experimental.pallas import tpu_sc as plsc

_N = 10000
_E = 160000
_D = 256
_DE = 16

_NC = 2
_NS = 16
_NW = _NC * _NS
_GW = 128

_R = 320
_WIN = 3200
_BSZ = 128
_ACCR = _R + 8
_DUMMY = _R + 4

_CP = pltpu.CompilerParams()
if "needs_layout_passes" in pltpu.CompilerParams.__dataclass_fields__:
    _CP = dataclasses.replace(_CP, needs_layout_passes=False)


def _sc_gather(table, src2, dst2):
    e = src2.shape[1]
    d = table.shape[1]
    mesh = plsc.VectorSubcoreMesh(core_axis_name="core", subcore_axis_name="subcore")

    @pl.kernel(
        out_type=(jax.ShapeDtypeStruct((e, d), table.dtype),
                  jax.ShapeDtypeStruct((e, d), table.dtype)),
        mesh=mesh,
    )
    def k(table_hbm, src_hbm, dst_hbm, gi_hbm, gj_hbm):
        def body(idx_v, out_v):
            pltpu.sync_copy(table_hbm.at[idx_v.at[0]], out_v)

        pipe = pltpu.emit_pipeline(
            body,
            grid=(e // _GW,),
            in_specs=[pl.BlockSpec((1, _GW), lambda i: (0, i))],
            out_specs=[pl.BlockSpec((_GW, d), lambda i: (i, 0))],
            core_axis_name=("core", "subcore"),
            dimension_semantics=(pltpu.PARALLEL,),
        )
        pipe(src_hbm, gi_hbm)
        pipe(dst_hbm, gj_hbm)

    return k(table, src2, dst2)


def _sc_scatter_add(messages, ridx):
    e, d = messages.shape
    nwin = e // _WIN
    last_r = _N - (_NW - 1) * _R
    mesh = plsc.VectorSubcoreMesh(core_axis_name="core", subcore_axis_name="subcore")

    @pl.kernel(
        out_type=jax.ShapeDtypeStruct((_N, d), jnp.float32),
        mesh=mesh,
        compiler_params=_CP,
        scratch_types=[
            pltpu.VMEM((_ACCR, d), jnp.float32),
            pltpu.VMEM((_WIN,), jnp.int32),
            pltpu.VMEM((_WIN,), jnp.int32),
            pltpu.VMEM((_WIN + 2 * _BSZ,), jnp.int32),
            pltpu.VMEM((_WIN + 2 * _BSZ,), jnp.int32),
            pltpu.VMEM((_BSZ,), jnp.int32),
            pltpu.VMEM((_BSZ, d), jnp.float32),
            pltpu.SemaphoreType.DMA,
            pltpu.SemaphoreType.DMA,
        ],
    )
    def k(msg_hbm, ridx_hbm, out_hbm, acc_v, widxa_v, widxb_v, ids_v, rel_v,
          idsrow_v, gbuf_v, gsem, wsem):
        c = lax.axis_index("core")
        s = lax.axis_index("subcore")
        w = c * _NS + s
        base = w * _R
        my_r = jnp.where(w == _NW - 1, last_r, _R)
        iota16 = lax.iota(jnp.int32, 16)
        zeros16f = jnp.zeros((16,), jnp.float32)

        @pl.loop(0, _ACCR)
        def _(r):
            for j in range(d // 16):
                acc_v[r, pl.ds(j * 16, 16)] = zeros16f

        def process_window(w0, widx_v):
            def scan_body(g, cnt):
                v = widx_v[pl.ds(g * 16, 16)]
                rel = v - base
                m = (rel >= 0) & (rel < my_r)
                eid = jnp.full((16,), w0 + g * 16, jnp.int32) + iota16
                plsc.store_compressed(ids_v.at[pl.ds(cnt, 16)], eid, mask=m)
                plsc.store_compressed(rel_v.at[pl.ds(cnt, 16)], rel, mask=m)
                pc = plsc.all_reduce_population_count(m)
                return cnt + pc[0]

            cnt = lax.fori_loop(0, _WIN // 16, scan_body, jnp.int32(0))

            for t in range(_BSZ // 16):
                ids_v[pl.ds(cnt + t * 16, 16)] = jnp.zeros((16,), jnp.int32)
                rel_v[pl.ds(cnt + t * 16, 16)] = jnp.full((16,), _DUMMY,
                                                          jnp.int32)

            nb = (cnt + _BSZ - 1) // _BSZ

            def batch_body(b, carry):
                for j in range(_BSZ // 16):
                    idsrow_v[pl.ds(j * 16, 16)] = (
                        ids_v[pl.ds(b * _BSZ + j * 16, 16)])
                pltpu.async_copy(msg_hbm.at[idsrow_v], gbuf_v, gsem).wait()

                @pl.loop(0, _BSZ // 16)
                def _(g):
                    relv = rel_v[pl.ds(b * _BSZ + g * 16, 16)]
                    for kk in range(16):
                        rr = relv[kk]
                        grow = g * 16 + kk
                        for j in range(d // 16):
                            sl = pl.ds(j * 16, 16)
                            acc_v[rr, sl] = acc_v[rr, sl] + gbuf_v[grow, sl]
                return carry

            lax.fori_loop(0, nb, batch_body, jnp.int32(0))

        wbytes = pl.ds(0, _WIN)

        def wwait():
            pltpu.make_async_copy(ridx_hbm.at[wbytes], widxa_v, wsem).wait()

        pltpu.async_copy(ridx_hbm.at[pl.ds(0, _WIN)], widxa_v, wsem)

        @pl.loop(0, nwin // 2)
        def _(t):
            w0 = 2 * t * _WIN
            wwait()

            @pl.when(2 * t + 1 < nwin)
            def _():
                pltpu.async_copy(ridx_hbm.at[pl.ds(w0 + _WIN, _WIN)],
                                 widxb_v, wsem)

            process_window(w0, widxa_v)
            wwait()

            @pl.when(2 * t + 2 < nwin)
            def _():
                pltpu.async_copy(ridx_hbm.at[pl.ds(w0 + 2 * _WIN, _WIN)],
                                 widxa_v, wsem)

            process_window(w0 + _WIN, widxb_v)

        @pl.when(w < _NW - 1)
        def _():
            pltpu.sync_copy(acc_v.at[pl.ds(0, _R)],
                            out_hbm.at[pl.ds(base, _R)])

        @pl.when(w == _NW - 1)
        def _():
            pltpu.sync_copy(acc_v.at[pl.ds(0, last_r)],
                            out_hbm.at[pl.ds((_NW - 1) * _R, last_r)])

    return k(messages, ridx)


def _gelu(x):
    return x * 0.5 * (1.0 + lax.erf(x * (1.0 / math.sqrt(2.0))))


def _dot(a, b):
    return lax.dot_general(a, b, (((1,), (0,)), ((), ())),
                           precision=lax.Precision.HIGHEST,
                           preferred_element_type=jnp.float32)


def _tc_msg_mlp(gi, gj, ev, W1, b1, W2, b2):
    e = gi.shape[0]
    be = 640
    hm = W1.shape[1]

    def body(gi_ref, gj_ref, ev_ref, W1_ref, b1_ref, W2_ref, b2_ref, out_ref):
        x = (_dot(gi_ref[...], W1_ref[0:_D, :])
             + _dot(gj_ref[...], W1_ref[_D:2 * _D, :])
             + _dot(ev_ref[...], W1_ref[2 * _D:2 * _D + _DE, :])
             + b1_ref[...])
        h = _gelu(x)
        out_ref[...] = _gelu(_dot(h, W2_ref[...]) + b2_ref[...])

    return pl.pallas_call(
        body,
        grid=(e // be,),
        in_specs=[pl.BlockSpec((be, _D), lambda i: (i, 0)),
                  pl.BlockSpec((be, _D), lambda i: (i, 0)),
                  pl.BlockSpec((be, _DE), lambda i: (i, 0)),
                  pl.BlockSpec(W1.shape, lambda i: (0, 0)),
                  pl.BlockSpec((1, hm), lambda i: (0, 0)),
                  pl.BlockSpec(W2.shape, lambda i: (0, 0)),
                  pl.BlockSpec((1, _D), lambda i: (0, 0))],
        out_specs=pl.BlockSpec((be, _D), lambda i: (i, 0)),
        out_shape=jax.ShapeDtypeStruct((e, _D), jnp.float32),
    )(gi, gj, ev, W1, b1.reshape(1, -1), W2, b2.reshape(1, -1))


def _tc_combine(node, boxes, Wc1, bc1, Wc2, bc2):
    n = node.shape[0]
    bn = 400
    hc = Wc1.shape[1]

    def body(x_ref, m_ref, Wc1_ref, bc1_ref, Wc2_ref, bc2_ref, out_ref):
        x = x_ref[...]
        cpre = (_dot(x, Wc1_ref[0:_D, :])
                + _dot(m_ref[...], Wc1_ref[_D:2 * _D, :])
                + bc1_ref[...])
        c1 = _gelu(cpre)
        out_ref[...] = x + _gelu(_dot(c1, Wc2_ref[...]) + bc2_ref[...])

    return pl.pallas_call(
        body,
        grid=(n // bn,),
        in_specs=[pl.BlockSpec((bn, _D), lambda i: (i, 0)),
                  pl.BlockSpec((bn, _D), lambda i: (i, 0)),
                  pl.BlockSpec(Wc1.shape, lambda i: (0, 0)),
                  pl.BlockSpec((1, hc), lambda i: (0, 0)),
                  pl.BlockSpec(Wc2.shape, lambda i: (0, 0)),
                  pl.BlockSpec((1, _D), lambda i: (0, 0))],
        out_specs=pl.BlockSpec((bn, _D), lambda i: (i, 0)),
        out_shape=jax.ShapeDtypeStruct((n, _D), jnp.float32),
    )(node, boxes, Wc1, bc1.reshape(1, -1), Wc2, bc2.reshape(1, -1))


def kernel(node_vectors, edge_vectors, edge_indices, W1, b1, W2, b2,
           Wc1, bc1, Wc2, bc2):
    src = edge_indices[:, 0].reshape(1, _E)
    dst = edge_indices[:, 1]
    gi, gj = _sc_gather(node_vectors, src, dst.reshape(1, _E))
    messages = _tc_msg_mlp(gi, gj, edge_vectors, W1, b1, W2, b2)
    boxes = _sc_scatter_add(messages, dst)
    return _tc_combine(node_vectors, boxes, Wc1, bc1, Wc2, bc2)


# speedup vs baseline: 1.1262x; 1.1262x over previous
import dataclasses
import functools
import math

import jax
import jax.numpy as jnp
from jax import lax
from jax.experimental import pallas as pl
from jax.experimental.pallas import tpu as pltpu
from jax.experimental.pallas import tpu_sc as plsc

_N = 10000
_E = 160000
_D = 256
_DE = 16

_NC = 2
_NS = 16
_NW = _NC * _NS
_GW = 128

_R = 320
_WIN = 3200
_BSZ = 64
_ACCR = _R + 8
_DUMMY = _R + 4

_CP = pltpu.CompilerParams()
if "needs_layout_passes" in pltpu.CompilerParams.__dataclass_fields__:
    _CP = dataclasses.replace(_CP, needs_layout_passes=False)


def _sc_gather(table, src2, dst2):
    e = src2.shape[1]
    d = table.shape[1]
    mesh = plsc.VectorSubcoreMesh(core_axis_name="core", subcore_axis_name="subcore")

    @pl.kernel(
        out_type=(jax.ShapeDtypeStruct((e, d), table.dtype),
                  jax.ShapeDtypeStruct((e, d), table.dtype)),
        mesh=mesh,
    )
    def k(table_hbm, src_hbm, dst_hbm, gi_hbm, gj_hbm):
        def body(idx_v, out_v):
            pltpu.sync_copy(table_hbm.at[idx_v.at[0]], out_v)

        pipe = pltpu.emit_pipeline(
            body,
            grid=(e // _GW,),
            in_specs=[pl.BlockSpec((1, _GW), lambda i: (0, i))],
            out_specs=[pl.BlockSpec((_GW, d), lambda i: (i, 0))],
            core_axis_name=("core", "subcore"),
            dimension_semantics=(pltpu.PARALLEL,),
        )
        pipe(src_hbm, gi_hbm)
        pipe(dst_hbm, gj_hbm)

    return k(table, src2, dst2)


def _sc_scatter_add(messages, ridx):
    e, d = messages.shape
    nwin = e // _WIN
    last_r = _N - (_NW - 1) * _R
    mesh = plsc.VectorSubcoreMesh(core_axis_name="core", subcore_axis_name="subcore")

    @pl.kernel(
        out_type=jax.ShapeDtypeStruct((_N, d), jnp.float32),
        mesh=mesh,
        compiler_params=_CP,
        scratch_types=[
            pltpu.VMEM((_ACCR, d), jnp.float32),
            pltpu.VMEM((_WIN,), jnp.int32),
            pltpu.VMEM((_WIN,), jnp.int32),
            pltpu.VMEM((_WIN + 2 * _BSZ,), jnp.int32),
            pltpu.VMEM((_WIN + 2 * _BSZ,), jnp.int32),
            pltpu.VMEM((_BSZ, d), jnp.float32),
            pltpu.VMEM((_BSZ, d), jnp.float32),
            pltpu.SemaphoreType.DMA,
            pltpu.SemaphoreType.DMA,
        ],
    )
    def k(msg_hbm, ridx_hbm, out_hbm, acc_v, widxa_v, widxb_v, ids_v, rel_v,
          gbufa_v, gbufb_v, gsem, wsem):
        c = lax.axis_index("core")
        s = lax.axis_index("subcore")
        w = c * _NS + s
        base = w * _R
        my_r = jnp.where(w == _NW - 1, last_r, _R)
        iota16 = lax.iota(jnp.int32, 16)
        zeros16f = jnp.zeros((16,), jnp.float32)

        @pl.loop(0, _ACCR)
        def _(r):
            for j in range(d // 16):
                acc_v[r, pl.ds(j * 16, 16)] = zeros16f

        def process_window(w0, widx_v):
            def scan_body(g, cnt):
                v = widx_v[pl.ds(g * 16, 16)]
                rel = v - base
                m = (rel >= 0) & (rel < my_r)
                eid = jnp.full((16,), w0 + g * 16, jnp.int32) + iota16
                plsc.store_compressed(ids_v.at[pl.ds(cnt, 16)], eid, mask=m)
                plsc.store_compressed(rel_v.at[pl.ds(cnt, 16)], rel, mask=m)
                pc = plsc.all_reduce_population_count(m)
                return cnt + pc[0]

            cnt = lax.fori_loop(0, _WIN // 16, scan_body, jnp.int32(0))

            for t in range(_BSZ // 16):
                ids_v[pl.ds(cnt + t * 16, 16)] = jnp.zeros((16,), jnp.int32)
                rel_v[pl.ds(cnt + t * 16, 16)] = jnp.full((16,), _DUMMY,
                                                          jnp.int32)

            nb = (cnt + _BSZ - 1) // _BSZ

            def fire(b, gbuf):
                pltpu.async_copy(
                    msg_hbm.at[ids_v.at[pl.ds(b * _BSZ, _BSZ)]], gbuf, gsem)

            def drain(b, gbuf):
                pltpu.make_async_copy(
                    msg_hbm.at[ids_v.at[pl.ds(b * _BSZ, _BSZ)]],
                    gbuf, gsem).wait()

            def accumulate(b, gbuf):
                ng = (cnt - b * _BSZ + 15) // 16
                ngc = jnp.minimum(ng, _BSZ // 16)

                @pl.loop(0, ngc)
                def _(g):
                    relv = rel_v[pl.ds(b * _BSZ + g * 16, 16)]
                    for kk in range(16):
                        rr = relv[kk]
                        loads = []
                        for j in range(d // 16):
                            sl = pl.ds(j * 16, 16)
                            loads.append((sl, acc_v[rr, sl],
                                          gbuf[g * 16 + kk, sl]))
                        for sl, a, g2 in loads:
                            acc_v[rr, sl] = a + g2

            @pl.when(nb > 0)
            def _():
                fire(jnp.int32(0), gbufa_v)

            @pl.loop(0, (nb + 1) // 2)
            def _(t):
                b0 = 2 * t

                @pl.when(b0 + 1 < nb)
                def _():
                    fire(b0 + 1, gbufb_v)

                drain(b0, gbufa_v)
                accumulate(b0, gbufa_v)

                @pl.when(b0 + 2 < nb)
                def _():
                    fire(b0 + 2, gbufa_v)

                @pl.when(b0 + 1 < nb)
                def _():
                    drain(b0 + 1, gbufb_v)
                    accumulate(b0 + 1, gbufb_v)

        wbytes = pl.ds(0, _WIN)

        def wwait():
            pltpu.make_async_copy(ridx_hbm.at[wbytes], widxa_v, wsem).wait()

        pltpu.async_copy(ridx_hbm.at[pl.ds(0, _WIN)], widxa_v, wsem)

        @pl.loop(0, nwin // 2)
        def _(t):
            w0 = 2 * t * _WIN
            wwait()

            @pl.when(2 * t + 1 < nwin)
            def _():
                pltpu.async_copy(ridx_hbm.at[pl.ds(w0 + _WIN, _WIN)],
                                 widxb_v, wsem)

            process_window(w0, widxa_v)
            wwait()

            @pl.when(2 * t + 2 < nwin)
            def _():
                pltpu.async_copy(ridx_hbm.at[pl.ds(w0 + 2 * _WIN, _WIN)],
                                 widxa_v, wsem)

            process_window(w0 + _WIN, widxb_v)

        @pl.when(w < _NW - 1)
        def _():
            pltpu.sync_copy(acc_v.at[pl.ds(0, _R)],
                            out_hbm.at[pl.ds(base, _R)])

        @pl.when(w == _NW - 1)
        def _():
            pltpu.sync_copy(acc_v.at[pl.ds(0, last_r)],
                            out_hbm.at[pl.ds((_NW - 1) * _R, last_r)])

    return k(messages, ridx)


def _gelu(x):
    return x * 0.5 * (1.0 + lax.erf(x * (1.0 / math.sqrt(2.0))))


def _dot(a, b):
    return lax.dot_general(a, b, (((1,), (0,)), ((), ())),
                           precision=lax.Precision.HIGHEST,
                           preferred_element_type=jnp.float32)


def _tc_msg_mlp(gi, gj, ev, W1, b1, W2, b2):
    e = gi.shape[0]
    be = 640
    hm = W1.shape[1]

    def body(gi_ref, gj_ref, ev_ref, W1_ref, b1_ref, W2_ref, b2_ref, out_ref):
        x = (_dot(gi_ref[...], W1_ref[0:_D, :])
             + _dot(gj_ref[...], W1_ref[_D:2 * _D, :])
             + _dot(ev_ref[...], W1_ref[2 * _D:2 * _D + _DE, :])
             + b1_ref[...])
        h = _gelu(x)
        out_ref[...] = _gelu(_dot(h, W2_ref[...]) + b2_ref[...])

    return pl.pallas_call(
        body,
        grid=(e // be,),
        in_specs=[pl.BlockSpec((be, _D), lambda i: (i, 0)),
                  pl.BlockSpec((be, _D), lambda i: (i, 0)),
                  pl.BlockSpec((be, _DE), lambda i: (i, 0)),
                  pl.BlockSpec(W1.shape, lambda i: (0, 0)),
                  pl.BlockSpec((1, hm), lambda i: (0, 0)),
                  pl.BlockSpec(W2.shape, lambda i: (0, 0)),
                  pl.BlockSpec((1, _D), lambda i: (0, 0))],
        out_specs=pl.BlockSpec((be, _D), lambda i: (i, 0)),
        out_shape=jax.ShapeDtypeStruct((e, _D), jnp.float32),
    )(gi, gj, ev, W1, b1.reshape(1, -1), W2, b2.reshape(1, -1))


def _tc_combine(node, boxes, Wc1, bc1, Wc2, bc2):
    n = node.shape[0]
    bn = 400
    hc = Wc1.shape[1]

    def body(x_ref, m_ref, Wc1_ref, bc1_ref, Wc2_ref, bc2_ref, out_ref):
        x = x_ref[...]
        cpre = (_dot(x, Wc1_ref[0:_D, :])
                + _dot(m_ref[...], Wc1_ref[_D:2 * _D, :])
                + bc1_ref[...])
        c1 = _gelu(cpre)
        out_ref[...] = x + _gelu(_dot(c1, Wc2_ref[...]) + bc2_ref[...])

    return pl.pallas_call(
        body,
        grid=(n // bn,),
        in_specs=[pl.BlockSpec((bn, _D), lambda i: (i, 0)),
                  pl.BlockSpec((bn, _D), lambda i: (i, 0)),
                  pl.BlockSpec(Wc1.shape, lambda i: (0, 0)),
                  pl.BlockSpec((1, hc), lambda i: (0, 0)),
                  pl.BlockSpec(Wc2.shape, lambda i: (0, 0)),
                  pl.BlockSpec((1, _D), lambda i: (0, 0))],
        out_specs=pl.BlockSpec((bn, _D), lambda i: (i, 0)),
        out_shape=jax.ShapeDtypeStruct((n, _D), jnp.float32),
    )(node, boxes, Wc1, bc1.reshape(1, -1), Wc2, bc2.reshape(1, -1))


def kernel(node_vectors, edge_vectors, edge_indices, W1, b1, W2, b2,
           Wc1, bc1, Wc2, bc2):
    src = edge_indices[:, 0].reshape(1, _E)
    dst = edge_indices[:, 1]
    gi, gj = _sc_gather(node_vectors, src, dst.reshape(1, _E))
    messages = _tc_msg_mlp(gi, gj, edge_vectors, W1, b1, W2, b2)
    boxes = _sc_scatter_add(messages, dst)
    return _tc_combine(node_vectors, boxes, Wc1, bc1, Wc2, bc2)


# speedup vs baseline: 1.4097x; 1.2517x over previous
import dataclasses
import functools
import math

import jax
import jax.numpy as jnp
from jax import lax
from jax.experimental import pallas as pl
from jax.experimental.pallas import tpu as pltpu
from jax.experimental.pallas import tpu_sc as plsc

_N = 10000
_E = 160000
_D = 256
_DE = 16

_NC = 2
_NS = 16
_NW = _NC * _NS
_GW = 128

_R = 320
_WIN = 3200
_BSZ = 64
_ACCR = _R + 8
_DUMMY = _R + 4

_CP = pltpu.CompilerParams()
if "needs_layout_passes" in pltpu.CompilerParams.__dataclass_fields__:
    _CP = dataclasses.replace(_CP, needs_layout_passes=False)


def _sc_gather(table_p, table_q, src2, dst2):
    e = src2.shape[1]
    d = table_p.shape[1]
    mesh = plsc.VectorSubcoreMesh(core_axis_name="core", subcore_axis_name="subcore")

    @pl.kernel(
        out_type=(jax.ShapeDtypeStruct((e, d), table_p.dtype),
                  jax.ShapeDtypeStruct((e, d), table_p.dtype)),
        mesh=mesh,
    )
    def k(tp_hbm, tq_hbm, src_hbm, dst_hbm, gi_hbm, gj_hbm):
        def mkbody(table_hbm):
            def body(idx_v, out_v):
                pltpu.sync_copy(table_hbm.at[idx_v.at[0]], out_v)
            return body

        def pipe(table_hbm, idx_hbm, out_hbm):
            pltpu.emit_pipeline(
                mkbody(table_hbm),
                grid=(e // _GW,),
                in_specs=[pl.BlockSpec((1, _GW), lambda i: (0, i))],
                out_specs=[pl.BlockSpec((_GW, d), lambda i: (i, 0))],
                core_axis_name=("core", "subcore"),
                dimension_semantics=(pltpu.PARALLEL,),
            )(idx_hbm, out_hbm)

        pipe(tp_hbm, src_hbm, gi_hbm)
        pipe(tq_hbm, dst_hbm, gj_hbm)

    return k(table_p, table_q, src2, dst2)


def _tc_pq(node, W1ab_p):
    n = node.shape[0]
    bn = 400
    dd = W1ab_p.shape[1]

    def body(x_ref, w_ref, out_ref):
        out_ref[...] = _dot(x_ref[...], w_ref[...])

    return pl.pallas_call(
        body,
        grid=(n // bn,),
        in_specs=[pl.BlockSpec((bn, _D), lambda i: (i, 0)),
                  pl.BlockSpec(W1ab_p.shape, lambda i: (0, 0))],
        out_specs=pl.BlockSpec((bn, dd), lambda i: (i, 0)),
        out_shape=jax.ShapeDtypeStruct((n, dd), jnp.float32),
    )(node, W1ab_p)


def _sc_scatter_add(messages, ridx):
    e, d = messages.shape
    nwin = e // _WIN
    last_r = _N - (_NW - 1) * _R
    mesh = plsc.VectorSubcoreMesh(core_axis_name="core", subcore_axis_name="subcore")

    @pl.kernel(
        out_type=jax.ShapeDtypeStruct((_N, d), jnp.float32),
        mesh=mesh,
        compiler_params=_CP,
        scratch_types=[
            pltpu.VMEM((_ACCR, d), jnp.float32),
            pltpu.VMEM((_WIN,), jnp.int32),
            pltpu.VMEM((_WIN,), jnp.int32),
            pltpu.VMEM((_WIN + 2 * _BSZ,), jnp.int32),
            pltpu.VMEM((_WIN + 2 * _BSZ,), jnp.int32),
            pltpu.VMEM((_BSZ, d), jnp.float32),
            pltpu.VMEM((_BSZ, d), jnp.float32),
            pltpu.SemaphoreType.DMA,
            pltpu.SemaphoreType.DMA,
        ],
    )
    def k(msg_hbm, ridx_hbm, out_hbm, acc_v, widxa_v, widxb_v, ids_v, rel_v,
          gbufa_v, gbufb_v, gsem, wsem):
        c = lax.axis_index("core")
        s = lax.axis_index("subcore")
        w = c * _NS + s
        base = w * _R
        my_r = jnp.where(w == _NW - 1, last_r, _R)
        iota16 = lax.iota(jnp.int32, 16)
        zeros16f = jnp.zeros((16,), jnp.float32)

        @pl.loop(0, _ACCR)
        def _(r):
            for j in range(d // 16):
                acc_v[r, pl.ds(j * 16, 16)] = zeros16f

        def process_window(w0, widx_v):
            def scan_body(g, cnt):
                v = widx_v[pl.ds(g * 16, 16)]
                rel = v - base
                m = (rel >= 0) & (rel < my_r)
                eid = jnp.full((16,), w0 + g * 16, jnp.int32) + iota16
                plsc.store_compressed(ids_v.at[pl.ds(cnt, 16)], eid, mask=m)
                plsc.store_compressed(rel_v.at[pl.ds(cnt, 16)], rel, mask=m)
                pc = plsc.all_reduce_population_count(m)
                return cnt + pc[0]

            cnt = lax.fori_loop(0, _WIN // 16, scan_body, jnp.int32(0))

            for t in range(_BSZ // 16):
                ids_v[pl.ds(cnt + t * 16, 16)] = jnp.zeros((16,), jnp.int32)
                rel_v[pl.ds(cnt + t * 16, 16)] = jnp.full((16,), _DUMMY,
                                                          jnp.int32)

            nb = (cnt + _BSZ - 1) // _BSZ

            def fire(b, gbuf):
                pltpu.async_copy(
                    msg_hbm.at[ids_v.at[pl.ds(b * _BSZ, _BSZ)]], gbuf, gsem)

            def drain(b, gbuf):
                pltpu.make_async_copy(
                    msg_hbm.at[ids_v.at[pl.ds(b * _BSZ, _BSZ)]],
                    gbuf, gsem).wait()

            def accumulate(b, gbuf):
                ng = (cnt - b * _BSZ + 15) // 16
                ngc = jnp.minimum(ng, _BSZ // 16)

                @pl.loop(0, ngc)
                def _(g):
                    relv = rel_v[pl.ds(b * _BSZ + g * 16, 16)]
                    for kk in range(16):
                        rr = relv[kk]
                        loads = []
                        for j in range(d // 16):
                            sl = pl.ds(j * 16, 16)
                            loads.append((sl, acc_v[rr, sl],
                                          gbuf[g * 16 + kk, sl]))
                        for sl, a, g2 in loads:
                            acc_v[rr, sl] = a + g2

            @pl.when(nb > 0)
            def _():
                fire(jnp.int32(0), gbufa_v)

            @pl.loop(0, (nb + 1) // 2)
            def _(t):
                b0 = 2 * t

                @pl.when(b0 + 1 < nb)
                def _():
                    fire(b0 + 1, gbufb_v)

                drain(b0, gbufa_v)
                accumulate(b0, gbufa_v)

                @pl.when(b0 + 2 < nb)
                def _():
                    fire(b0 + 2, gbufa_v)

                @pl.when(b0 + 1 < nb)
                def _():
                    drain(b0 + 1, gbufb_v)
                    accumulate(b0 + 1, gbufb_v)

        wbytes = pl.ds(0, _WIN)

        def wwait():
            pltpu.make_async_copy(ridx_hbm.at[wbytes], widxa_v, wsem).wait()

        pltpu.async_copy(ridx_hbm.at[pl.ds(0, _WIN)], widxa_v, wsem)

        @pl.loop(0, nwin // 2)
        def _(t):
            w0 = 2 * t * _WIN
            wwait()

            @pl.when(2 * t + 1 < nwin)
            def _():
                pltpu.async_copy(ridx_hbm.at[pl.ds(w0 + _WIN, _WIN)],
                                 widxb_v, wsem)

            process_window(w0, widxa_v)
            wwait()

            @pl.when(2 * t + 2 < nwin)
            def _():
                pltpu.async_copy(ridx_hbm.at[pl.ds(w0 + 2 * _WIN, _WIN)],
                                 widxa_v, wsem)

            process_window(w0 + _WIN, widxb_v)

        @pl.when(w < _NW - 1)
        def _():
            pltpu.sync_copy(acc_v.at[pl.ds(0, _R)],
                            out_hbm.at[pl.ds(base, _R)])

        @pl.when(w == _NW - 1)
        def _():
            pltpu.sync_copy(acc_v.at[pl.ds(0, last_r)],
                            out_hbm.at[pl.ds((_NW - 1) * _R, last_r)])

    return k(messages, ridx)


def _gelu(x):
    return x * 0.5 * (1.0 + lax.erf(x * (1.0 / math.sqrt(2.0))))


def _dot(a, b):
    return lax.dot_general(a, b, (((1,), (0,)), ((), ())),
                           precision=lax.Precision.HIGHEST,
                           preferred_element_type=jnp.float32)


def _tc_msg_mlp(gi, gj, ev, W1c_p, b1_p, W2_p, b2):
    e = gi.shape[0]
    be = 640
    hm = W1c_p.shape[1]

    def body(gi_ref, gj_ref, ev_ref, W1c_ref, b1_ref, W2_ref, b2_ref,
             out_ref):
        x = (gi_ref[...] + gj_ref[...]
             + _dot(ev_ref[...], W1c_ref[...]) + b1_ref[...])
        h = _gelu(x)
        out_ref[...] = _gelu(_dot(h, W2_ref[...]) + b2_ref[...])

    return pl.pallas_call(
        body,
        grid=(e // be,),
        in_specs=[pl.BlockSpec((be, hm), lambda i: (i, 0)),
                  pl.BlockSpec((be, hm), lambda i: (i, 0)),
                  pl.BlockSpec((be, _DE), lambda i: (i, 0)),
                  pl.BlockSpec(W1c_p.shape, lambda i: (0, 0)),
                  pl.BlockSpec((1, hm), lambda i: (0, 0)),
                  pl.BlockSpec(W2_p.shape, lambda i: (0, 0)),
                  pl.BlockSpec((1, _D), lambda i: (0, 0))],
        out_specs=pl.BlockSpec((be, _D), lambda i: (i, 0)),
        out_shape=jax.ShapeDtypeStruct((e, _D), jnp.float32),
    )(gi, gj, ev, W1c_p, b1_p.reshape(1, -1), W2_p, b2.reshape(1, -1))


def _tc_combine(node, boxes, Wc1, bc1, Wc2, bc2):
    n = node.shape[0]
    bn = 400
    hc = Wc1.shape[1]

    def body(x_ref, m_ref, Wc1_ref, bc1_ref, Wc2_ref, bc2_ref, out_ref):
        x = x_ref[...]
        cpre = (_dot(x, Wc1_ref[0:_D, :])
                + _dot(m_ref[...], Wc1_ref[_D:2 * _D, :])
                + bc1_ref[...])
        c1 = _gelu(cpre)
        out_ref[...] = x + _gelu(_dot(c1, Wc2_ref[...]) + bc2_ref[...])

    return pl.pallas_call(
        body,
        grid=(n // bn,),
        in_specs=[pl.BlockSpec((bn, _D), lambda i: (i, 0)),
                  pl.BlockSpec((bn, _D), lambda i: (i, 0)),
                  pl.BlockSpec(Wc1.shape, lambda i: (0, 0)),
                  pl.BlockSpec((1, hc), lambda i: (0, 0)),
                  pl.BlockSpec(Wc2.shape, lambda i: (0, 0)),
                  pl.BlockSpec((1, _D), lambda i: (0, 0))],
        out_specs=pl.BlockSpec((bn, _D), lambda i: (i, 0)),
        out_shape=jax.ShapeDtypeStruct((n, _D), jnp.float32),
    )(node, boxes, Wc1, bc1.reshape(1, -1), Wc2, bc2.reshape(1, -1))


def kernel(node_vectors, edge_vectors, edge_indices, W1, b1, W2, b2,
           Wc1, bc1, Wc2, bc2):
    src = edge_indices[:, 0].reshape(1, _E)
    dst = edge_indices[:, 1]
    hm = W1.shape[1]
    hmp = 384
    padc = ((0, 0), (0, hmp - hm))
    W1ab_p = jnp.concatenate(
        [jnp.pad(W1[:_D], padc), jnp.pad(W1[_D:2 * _D], padc)], axis=1)
    W1c_p = jnp.pad(W1[2 * _D:], padc)
    b1_p = jnp.pad(b1, (0, hmp - hm))
    W2_p = jnp.pad(W2, ((0, hmp - hm), (0, 0)))
    pq = _tc_pq(node_vectors, W1ab_p)
    gi, gj = _sc_gather(pq[:, :hmp], pq[:, hmp:], src, dst.reshape(1, _E))
    messages = _tc_msg_mlp(gi, gj, edge_vectors, W1c_p, b1_p, W2_p, b2)
    boxes = _sc_scatter_add(messages, dst)
    return _tc_combine(node_vectors, boxes, Wc1, bc1, Wc2, bc2)
